# baseline (speedup 1.0000x reference)
import functools

import jax
import jax.numpy as jnp
from jax import lax
from jax.experimental import pallas as pl
from jax.experimental.pallas import tpu as pltpu
from jax.experimental.pallas import tpu_sc as plsc

BATCH = 16384
DIM = 64
CHUNK = 128


@functools.lru_cache(maxsize=None)
def _build():
    info = plsc.get_sparse_core_info()
    nc, ns = info.num_cores, info.num_subcores
    nw = nc * ns
    b_per_w = BATCH // nw
    kch = b_per_w // CHUNK
    ngrp = b_per_w // 16
    o_rows = b_per_w // 2

    mesh = plsc.VectorSubcoreMesh(core_axis_name="c", subcore_axis_name="s")

    @functools.partial(
        pl.kernel,
        mesh=mesh,
        out_type=(
            jax.ShapeDtypeStruct((BATCH // 2, 2 * DIM), jnp.float32),
            jax.ShapeDtypeStruct((BATCH // 2, 2 * DIM), jnp.float32),
        ),
        scratch_types=[
            pltpu.VMEM((b_per_w,), jnp.int32),
            pltpu.VMEM((kch, CHUNK), jnp.int32),
            pltpu.VMEM((b_per_w, 2 * DIM), jnp.float32),
            pltpu.VMEM((o_rows, 2 * DIM), jnp.float32),
            pltpu.SemaphoreType.DMA,
        ],
        compiler_params=pltpu.CompilerParams(needs_layout_passes=False),
    )
    def emb_kernel(user_ids, item_ids, emb_user2, emb_item2, out_u, out_i,
                   idx_v, pidx_v, pairs_v, packed_v, sem):
        wid = lax.axis_index("s") * nc + lax.axis_index("c")
        base = wid * b_per_w
        lanes = lax.iota(jnp.int32, 16)
        dcol_base = (lanes & 1) * DIM

        def one_table(ids_hbm, table_hbm, out_hbm):
            pltpu.sync_copy(ids_hbm.at[pl.ds(base, b_per_w)], idx_v)
            for k in range(b_per_w // 16):
                ids16 = idx_v[pl.ds(k * 16, 16)]
                pidx_v[k // 8, pl.ds((k % 8) * 16, 16)] = ids16 >> 1
            cps = [
                pltpu.async_copy(
                    table_hbm.at[pidx_v.at[j]],
                    pairs_v.at[pl.ds(j * CHUNK, CHUNK)],
                    sem,
                )
                for j in range(kch)
            ]
            for cp in cps:
                cp.wait()

            def extract(g, carry):
                ids16 = idx_v[pl.ds(g * 16, 16)]
                srows = g * 16 + lanes
                scol = (ids16 & 1) * DIM
                drows = g * 8 + (lanes >> 1)
                for j in range(DIM):
                    vals = plsc.load_gather(pairs_v, [srows, scol + j])
                    plsc.store_scatter(packed_v, [drows, dcol_base + j], vals)
                return carry

            lax.fori_loop(0, ngrp, extract, 0)
            pltpu.sync_copy(packed_v, out_hbm.at[pl.ds(wid * o_rows, o_rows)])

        one_table(user_ids, emb_user2, out_u)
        one_table(item_ids, emb_item2, out_i)

    return emb_kernel


def kernel(user_ids, item_ids, emb_user, emb_item):
    emb_kernel = _build()
    up = emb_user.reshape(emb_user.shape[0] // 2, 2 * DIM)
    ip = emb_item.reshape(emb_item.shape[0] // 2, 2 * DIM)
    out_u, out_i = emb_kernel(user_ids, item_ids, up, ip)
    return (out_u.reshape(BATCH, DIM), out_i.reshape(BATCH, DIM))


# speedup vs baseline: 1.0012x; 1.0012x over previous
import functools

import jax
import jax.numpy as jnp
from jax import lax
from jax.experimental import pallas as pl
from jax.experimental.pallas import tpu as pltpu
from jax.experimental.pallas import tpu_sc as plsc

BATCH = 16384
DIM = 64
CHUNK = 128


@functools.lru_cache(maxsize=None)
def _build():
    info = plsc.get_sparse_core_info()
    nc, ns = info.num_cores, info.num_subcores
    nw = nc * ns
    b_per_w = BATCH // nw
    kch = b_per_w // CHUNK
    ngrp = b_per_w // 16
    o_rows = b_per_w // 2

    mesh = plsc.VectorSubcoreMesh(core_axis_name="c", subcore_axis_name="s")

    @functools.partial(
        pl.kernel,
        mesh=mesh,
        out_type=(
            jax.ShapeDtypeStruct((BATCH // 2, 2 * DIM), jnp.float32),
            jax.ShapeDtypeStruct((BATCH // 2, 2 * DIM), jnp.float32),
        ),
        scratch_types=[
            pltpu.VMEM((b_per_w,), jnp.int32),
            pltpu.VMEM((kch, CHUNK), jnp.int32),
            pltpu.VMEM((b_per_w, 2 * DIM), jnp.float32),
            pltpu.VMEM((o_rows, 2 * DIM), jnp.float32),
            pltpu.SemaphoreType.DMA,
        ],
        compiler_params=pltpu.CompilerParams(
            use_tc_tiling_on_sc=True, needs_layout_passes=False
        ),
    )
    def emb_kernel(user_ids, item_ids, emb_user2, emb_item2, out_u, out_i,
                   idx_v, pidx_v, pairs_v, packed_v, sem):
        wid = lax.axis_index("s") * nc + lax.axis_index("c")
        base = wid * b_per_w
        lanes = lax.iota(jnp.int32, 16)
        dcol_base = (lanes & 1) * DIM

        def one_table(ids_hbm, table_hbm, out_hbm):
            pltpu.sync_copy(ids_hbm.at[pl.ds(base, b_per_w)], idx_v)
            for k in range(b_per_w // 16):
                ids16 = idx_v[pl.ds(k * 16, 16)]
                pidx_v[k // 8, pl.ds((k % 8) * 16, 16)] = ids16 >> 1
            cps = [
                pltpu.async_copy(
                    table_hbm.at[pidx_v.at[j]],
                    pairs_v.at[pl.ds(j * CHUNK, CHUNK)],
                    sem,
                )
                for j in range(kch)
            ]
            for cp in cps:
                cp.wait()

            def extract(g, carry):
                ids16 = idx_v[pl.ds(g * 16, 16)]
                srows = g * 16 + lanes
                scol = (ids16 & 1) * DIM
                drows = g * 8 + (lanes >> 1)
                for j in range(DIM):
                    vals = plsc.load_gather(pairs_v, [srows, scol + j])
                    plsc.store_scatter(packed_v, [drows, dcol_base + j], vals)
                return carry

            lax.fori_loop(0, ngrp, extract, 0)
            pltpu.sync_copy(packed_v, out_hbm.at[pl.ds(wid * o_rows, o_rows)])

        one_table(user_ids, emb_user2, out_u)
        one_table(item_ids, emb_item2, out_i)

    return emb_kernel


def kernel(user_ids, item_ids, emb_user, emb_item):
    emb_kernel = _build()
    up = emb_user.reshape(emb_user.shape[0] // 2, 2 * DIM)
    ip = emb_item.reshape(emb_item.shape[0] // 2, 2 * DIM)
    out_u, out_i = emb_kernel(user_ids, item_ids, up, ip)
    return (out_u.reshape(BATCH, DIM), out_i.reshape(BATCH, DIM))


# speedup vs baseline: 1.5057x; 1.5039x over previous
import functools

import jax
import jax.numpy as jnp
from jax import lax
from jax.experimental import pallas as pl
from jax.experimental.pallas import tpu as pltpu
from jax.experimental.pallas import tpu_sc as plsc

BATCH = 16384
DIM = 64
NBUF = 8


@functools.lru_cache(maxsize=None)
def _build():
    info = plsc.get_sparse_core_info()
    nc, ns = info.num_cores, info.num_subcores
    nw = nc * ns
    b_per_w = BATCH // nw

    mesh = plsc.VectorSubcoreMesh(core_axis_name="c", subcore_axis_name="s")

    @functools.partial(
        pl.kernel,
        mesh=mesh,
        out_type=(
            jax.ShapeDtypeStruct((BATCH, DIM), jnp.float32),
            jax.ShapeDtypeStruct((BATCH, DIM), jnp.float32),
        ),
        scratch_types=[
            pltpu.VMEM((b_per_w,), jnp.int32),
            pltpu.VMEM((NBUF, 8, DIM), jnp.float32),
            pltpu.VMEM((b_per_w, DIM), jnp.float32),
            pltpu.SemaphoreType.DMA,
        ],
        compiler_params=pltpu.CompilerParams(
            use_tc_tiling_on_sc=True, needs_layout_passes=False
        ),
    )
    def emb_kernel(user_ids, item_ids, emb_user, emb_item, out_u, out_i,
                   idx_v, ring, rows_v, sem):
        wid = lax.axis_index("s") * nc + lax.axis_index("c")
        base = wid * b_per_w
        lanes = lax.iota(jnp.int32, 16)

        def id_at(k):
            vec = idx_v[pl.ds((k >> 4) << 4, 16)]
            return jnp.max(jnp.where(lanes == (k & 15), vec, -1))

        def one_table(ids_hbm, table_hbm, out_hbm):
            pltpu.sync_copy(ids_hbm.at[pl.ds(base, b_per_w)], idx_v)

            def fire(k, slot):
                g = (id_at(k) >> 3) * 8
                pltpu.async_copy(
                    table_hbm.at[pl.ds(g, 8)], ring.at[slot], sem
                )

            for k in range(NBUF):
                fire(k, k)

            def body(k, carry):
                pltpu.make_async_copy(
                    table_hbm.at[pl.ds(0, 8)], ring.at[0], sem
                ).wait()
                slot = lax.rem(k, NBUF)
                r = id_at(k) & 7
                for q in range(DIM // 16):
                    rows_v[k, pl.ds(q * 16, 16)] = (
                        ring[slot, r, pl.ds(q * 16, 16)]
                    )

                @pl.when(k + NBUF < b_per_w)
                def _():
                    fire(k + NBUF, slot)

                return carry

            lax.fori_loop(0, b_per_w, body, 0)
            pltpu.sync_copy(rows_v, out_hbm.at[pl.ds(base, b_per_w)])

        one_table(user_ids, emb_user, out_u)
        one_table(item_ids, emb_item, out_i)

    return emb_kernel


def kernel(user_ids, item_ids, emb_user, emb_item):
    emb_kernel = _build()
    return emb_kernel(user_ids, item_ids, emb_user, emb_item)


# speedup vs baseline: 2.0331x; 1.3503x over previous
import functools

import jax
import jax.numpy as jnp
from jax import lax
from jax.experimental import pallas as pl
from jax.experimental.pallas import tpu as pltpu
from jax.experimental.pallas import tpu_sc as plsc

BATCH = 16384
DIM = 64
NBUF = 8


@functools.lru_cache(maxsize=None)
def _build():
    info = plsc.get_sparse_core_info()
    nc, ns = info.num_cores, info.num_subcores
    nw = nc * ns
    b_per_w = BATCH // nw

    mesh = plsc.VectorSubcoreMesh(core_axis_name="c", subcore_axis_name="s")

    @functools.partial(
        pl.kernel,
        mesh=mesh,
        out_type=(
            jax.ShapeDtypeStruct((BATCH, DIM), jnp.float32),
            jax.ShapeDtypeStruct((BATCH, DIM), jnp.float32),
        ),
        scratch_types=[
            pltpu.VMEM((b_per_w,), jnp.int32),
            pltpu.VMEM((NBUF, 8, DIM), jnp.float32),
            pltpu.VMEM((b_per_w, DIM), jnp.float32),
            pltpu.SemaphoreType.DMA,
        ],
        compiler_params=pltpu.CompilerParams(
            use_tc_tiling_on_sc=True, needs_layout_passes=False
        ),
    )
    def emb_kernel(user_ids, item_ids, emb_user, emb_item, out_u, out_i,
                   idx_v, ring, rows_v, sem):
        wid = lax.axis_index("s") * nc + lax.axis_index("c")
        base = wid * b_per_w
        lanes = lax.iota(jnp.int32, 16)

        def id_at(k):
            vec = idx_v[pl.ds((k >> 4) << 4, 16)]
            return jnp.max(jnp.where(lanes == (k & 15), vec, -1))

        def one_table(ids_hbm, table_hbm, out_hbm):
            pltpu.sync_copy(ids_hbm.at[pl.ds(base, b_per_w)], idx_v)

            def fire(k, slot):
                pltpu.async_copy(
                    table_hbm.at[id_at(k) >> 3], ring.at[slot], sem
                )

            for k in range(NBUF):
                fire(k, k)

            def body(k, carry):
                pltpu.make_async_copy(
                    table_hbm.at[0], ring.at[0], sem
                ).wait()
                slot = lax.rem(k, NBUF)
                r = id_at(k) & 7
                for q in range(DIM // 16):
                    rows_v[k, pl.ds(q * 16, 16)] = (
                        ring[slot, r, pl.ds(q * 16, 16)]
                    )

                @pl.when(k + NBUF < b_per_w)
                def _():
                    fire(k + NBUF, slot)

                return carry

            lax.fori_loop(0, b_per_w, body, 0)
            pltpu.sync_copy(rows_v, out_hbm.at[pl.ds(base, b_per_w)])

        one_table(user_ids, emb_user, out_u)
        one_table(item_ids, emb_item, out_i)

    return emb_kernel


def kernel(user_ids, item_ids, emb_user, emb_item):
    emb_kernel = _build()
    up3 = emb_user.reshape(emb_user.shape[0] // 8, 8, DIM)
    ip3 = emb_item.reshape(emb_item.shape[0] // 8, 8, DIM)
    return emb_kernel(user_ids, item_ids, up3, ip3)


# speedup vs baseline: 2.2617x; 1.1124x over previous
import functools

import jax
import jax.numpy as jnp
from jax import lax
from jax.experimental import pallas as pl
from jax.experimental.pallas import tpu as pltpu
from jax.experimental.pallas import tpu_sc as plsc

BATCH = 16384
DIM = 64
NBUF = 16


@functools.lru_cache(maxsize=None)
def _build():
    info = plsc.get_sparse_core_info()
    nc, ns = info.num_cores, info.num_subcores
    nw = nc * ns
    b_per_w = BATCH // nw

    mesh = plsc.VectorSubcoreMesh(core_axis_name="c", subcore_axis_name="s")

    @functools.partial(
        pl.kernel,
        mesh=mesh,
        out_type=jax.ShapeDtypeStruct((BATCH, DIM), jnp.float32),
        scratch_types=[
            pltpu.VMEM((b_per_w,), jnp.int32),
            pltpu.VMEM((NBUF, 8, DIM), jnp.float32),
            pltpu.VMEM((b_per_w, DIM), jnp.float32),
            pltpu.SemaphoreType.DMA,
        ],
        compiler_params=pltpu.CompilerParams(
            use_tc_tiling_on_sc=True, needs_layout_passes=False
        ),
    )
    def gather_kernel(ids_hbm, table_hbm, out_hbm, idx_v, ring, rows_v, sem):
        wid = lax.axis_index("s") * nc + lax.axis_index("c")
        base = wid * b_per_w
        lanes = lax.iota(jnp.int32, 16)

        def id_at(k):
            vec = idx_v[pl.ds((k >> 4) << 4, 16)]
            return jnp.max(jnp.where(lanes == (k & 15), vec, -1))

        pltpu.sync_copy(ids_hbm.at[pl.ds(base, b_per_w)], idx_v)

        def fire(k, slot):
            pltpu.async_copy(table_hbm.at[id_at(k) >> 3], ring.at[slot], sem)

        for k in range(NBUF):
            fire(k, k)

        def body(k, carry):
            pltpu.make_async_copy(table_hbm.at[0], ring.at[0], sem).wait()
            slot = lax.rem(k, NBUF)
            r = id_at(k) & 7
            for q in range(DIM // 16):
                rows_v[k, pl.ds(q * 16, 16)] = ring[slot, r, pl.ds(q * 16, 16)]

            @pl.when(k + NBUF < b_per_w)
            def _():
                fire(k + NBUF, slot)

            return carry

        lax.fori_loop(0, b_per_w, body, 0)
        pltpu.sync_copy(rows_v, out_hbm.at[pl.ds(base, b_per_w)])

    return gather_kernel


def kernel(user_ids, item_ids, emb_user, emb_item):
    gather_kernel = _build()
    up3 = emb_user.reshape(emb_user.shape[0] // 8, 8, DIM)
    ip3 = emb_item.reshape(emb_item.shape[0] // 8, 8, DIM)
    item_emb = gather_kernel(item_ids, ip3)
    user_emb = gather_kernel(user_ids, up3)
    return (user_emb, item_emb)


# speedup vs baseline: 2.2851x; 1.0104x over previous
import functools

import jax
import jax.numpy as jnp
from jax import lax
from jax.experimental import pallas as pl
from jax.experimental.pallas import tpu as pltpu
from jax.experimental.pallas import tpu_sc as plsc

BATCH = 16384
DIM = 64
NBUF = 32


@functools.lru_cache(maxsize=None)
def _build():
    info = plsc.get_sparse_core_info()
    nc, ns = info.num_cores, info.num_subcores
    nw = nc * ns
    b_per_w = BATCH // nw

    mesh = plsc.VectorSubcoreMesh(core_axis_name="c", subcore_axis_name="s")

    @functools.partial(
        pl.kernel,
        mesh=mesh,
        out_type=jax.ShapeDtypeStruct((BATCH, DIM), jnp.float32),
        scratch_types=[
            pltpu.VMEM((b_per_w,), jnp.int32),
            pltpu.VMEM((NBUF, 8, DIM), jnp.float32),
            pltpu.VMEM((b_per_w, DIM), jnp.float32),
            pltpu.SemaphoreType.DMA,
        ],
        compiler_params=pltpu.CompilerParams(
            use_tc_tiling_on_sc=True, needs_layout_passes=False
        ),
    )
    def gather_kernel(ids_hbm, table_hbm, out_hbm, idx_v, ring, rows_v, sem):
        wid = lax.axis_index("s") * nc + lax.axis_index("c")
        base = wid * b_per_w
        lanes = lax.iota(jnp.int32, 16)

        def id_at(k):
            vec = idx_v[pl.ds((k >> 4) << 4, 16)]
            return jnp.max(jnp.where(lanes == (k & 15), vec, -1))

        pltpu.sync_copy(ids_hbm.at[pl.ds(base, b_per_w)], idx_v)

        def fire(k, slot):
            pltpu.async_copy(table_hbm.at[id_at(k) >> 3], ring.at[slot], sem)

        for k in range(NBUF):
            fire(k, k)

        def body(k, carry):
            pltpu.make_async_copy(table_hbm.at[0], ring.at[0], sem).wait()
            slot = lax.rem(k, NBUF)
            r = id_at(k) & 7
            for q in range(DIM // 16):
                rows_v[k, pl.ds(q * 16, 16)] = ring[slot, r, pl.ds(q * 16, 16)]

            @pl.when(k + NBUF < b_per_w)
            def _():
                fire(k + NBUF, slot)

            return carry

        lax.fori_loop(0, b_per_w, body, 0)
        pltpu.sync_copy(rows_v, out_hbm.at[pl.ds(base, b_per_w)])

    return gather_kernel


def kernel(user_ids, item_ids, emb_user, emb_item):
    gather_kernel = _build()
    up3 = emb_user.reshape(emb_user.shape[0] // 8, 8, DIM)
    ip3 = emb_item.reshape(emb_item.shape[0] // 8, 8, DIM)
    item_emb = gather_kernel(item_ids, ip3)
    user_emb = gather_kernel(user_ids, up3)
    return (user_emb, item_emb)
